# Initial kernel scaffold; baseline (speedup 1.0000x reference)
import functools

import jax
import jax.numpy as jnp
from jax import lax
from jax.experimental import pallas as pl
from jax.experimental.pallas import tpu as pltpu
from jax.experimental.pallas import tpu_sc as plsc

N = 10000
E = 320000
D = 128
G = 16
EPS = 1e-5

NC = 2
NS = 16
NW = NC * NS
EW = E // NW
CH = 80
NCHUNK = EW // CH
RPT = N // NS

_mesh = plsc.VectorSubcoreMesh(core_axis_name="c", subcore_axis_name="s")


def _deg_body(dst_hbm, w_hbm, zeros_hbm, out_hbm, dst_v, w_v, deg_sh):
    cid = lax.axis_index("c")
    sid = lax.axis_index("s")
    wid = cid * NS + sid

    @pl.when(sid == 0)
    def _():
        pltpu.sync_copy(zeros_hbm, deg_sh)

    plsc.subcore_barrier()

    pltpu.sync_copy(dst_hbm.at[wid], dst_v)
    pltpu.sync_copy(w_hbm.at[wid], w_v)

    def chunk(j, carry):
        pltpu.sync_copy(w_v.at[j], deg_sh.at[dst_v.at[j]], add=True)
        return carry

    lax.fori_loop(0, NCHUNK, chunk, 0)
    plsc.subcore_barrier()

    @pl.when(sid == 0)
    def _():
        pltpu.sync_copy(deg_sh, out_hbm.at[cid])


_deg_pass = functools.partial(
    pl.kernel,
    out_type=jax.ShapeDtypeStruct((NC, N), jnp.float32),
    mesh=_mesh,
    scratch_types=[
        pltpu.VMEM((NCHUNK, CH), jnp.int32),
        pltpu.VMEM((NCHUNK, CH), jnp.float32),
        pltpu.MemorySpace.VMEM_SHARED((N,), jnp.float32),
    ],
)(_deg_body)


def _prep_body(x_ref, w_ref, degt_ref, z_ref, zsl_ref, dis_ref):
    deg = degt_ref[:, 0:1] + degt_ref[:, 1:2] + 1.0
    safe = jnp.where(deg > 0.0, deg, 1.0)
    dis = jnp.where(deg > 0.0, lax.rsqrt(safe), 0.0)
    x = jnp.dot(x_ref[...], w_ref[...], preferred_element_type=jnp.float32)
    z = x * dis
    z_ref[...] = z
    zsl_ref[...] = z * dis
    dis_ref[...] = dis


_prep_pass = pl.pallas_call(
    _prep_body,
    out_shape=[
        jax.ShapeDtypeStruct((N, D), jnp.float32),
        jax.ShapeDtypeStruct((N, D), jnp.float32),
        jax.ShapeDtypeStruct((N, 1), jnp.float32),
    ],
)


def _edge_body(src_hbm, dst_hbm, w_hbm, z_hbm, dis_hbm, zeros_hbm, out_hbm,
               src_v, dst_v, w_v, dis_v, rows_v, s_v, conv_sh, sem):
    cid = lax.axis_index("c")
    sid = lax.axis_index("s")
    wid = cid * NS + sid

    pltpu.sync_copy(zeros_hbm.at[pl.ds(sid * RPT, RPT)],
                    conv_sh.at[pl.ds(sid * RPT, RPT)])
    pltpu.sync_copy(dis_hbm, dis_v)
    pltpu.sync_copy(src_hbm.at[wid], src_v)
    pltpu.sync_copy(dst_hbm.at[wid], dst_v)
    pltpu.sync_copy(w_hbm.at[wid], w_v)
    plsc.subcore_barrier()

    def chunk(j, carry):
        pltpu.async_copy(z_hbm.at[src_v.at[j]], rows_v, sem).wait()
        for v in range(CH // 16):
            sl = pl.ds(v * 16, 16)
            dstv = dst_v[j, sl]
            disg = plsc.load_gather(dis_v, [dstv])
            s_v[sl] = w_v[j, sl] * disg

        def rowloop(r0, carry2):
            for u in range(4):
                r = r0 * 4 + u
                sc = s_v[r]
                for cv in range(D // 16):
                    csl = pl.ds(cv * 16, 16)
                    rows_v[r, csl] = rows_v[r, csl] * sc
            return carry2

        lax.fori_loop(0, CH // 4, rowloop, 0)
        pltpu.sync_copy(rows_v, conv_sh.at[dst_v.at[j]], add=True)
        return carry

    lax.fori_loop(0, NCHUNK, chunk, 0)
    plsc.subcore_barrier()

    pltpu.sync_copy(conv_sh.at[pl.ds(sid * RPT, RPT)],
                    out_hbm.at[cid].at[pl.ds(sid * RPT, RPT)])


_edge_pass = functools.partial(
    pl.kernel,
    out_type=jax.ShapeDtypeStruct((NC, N, D), jnp.float32),
    mesh=_mesh,
    scratch_types=[
        pltpu.VMEM((NCHUNK, CH), jnp.int32),
        pltpu.VMEM((NCHUNK, CH), jnp.int32),
        pltpu.VMEM((NCHUNK, CH), jnp.float32),
        pltpu.VMEM((N,), jnp.float32),
        pltpu.VMEM((CH, D), jnp.float32),
        pltpu.VMEM((CH,), jnp.float32),
        pltpu.MemorySpace.VMEM_SHARED((N, D), jnp.float32),
        pltpu.SemaphoreType.DMA,
    ],
)(_edge_body)


def _final_body(convp_ref, zsl_ref, b_ref, gnw_ref, gnb_ref, gnms_ref,
                batchr_ref, batchc_ref, h_ref, flat_ref):
    conv = convp_ref[0] + convp_ref[1] + zsl_ref[...] + b_ref[...]
    ms = gnms_ref[...]

    gids = lax.broadcasted_iota(jnp.int32, (G, N), 0)
    onehot = (batchr_ref[...] == gids).astype(jnp.float32)
    counts = jnp.sum(onehot, axis=1, keepdims=True)
    counts = jnp.maximum(counts, 1.0)
    s1 = jnp.dot(onehot, conv, preferred_element_type=jnp.float32)
    s2 = jnp.dot(onehot, conv * conv, preferred_element_type=jnp.float32)
    mean = s1 / counts
    msq = s2 / counts
    var = msq - (2.0 * ms - ms * ms) * mean * mean
    rstd = lax.rsqrt(var + EPS)
    scale = gnw_ref[...] * rstd
    shift = gnb_ref[...] - ms * mean * scale

    gcols = lax.broadcasted_iota(jnp.int32, (N, G), 1)
    onehot_t = (batchc_ref[...] == gcols).astype(jnp.float32)
    scale_full = jnp.dot(onehot_t, scale, preferred_element_type=jnp.float32)
    shift_full = jnp.dot(onehot_t, shift, preferred_element_type=jnp.float32)
    h = jnp.maximum(conv * scale_full + shift_full, 0.0)
    h_ref[...] = h

    neg_inf = jnp.float32(-jnp.inf)
    for g in range(G):
        mask = onehot_t[:, g:g + 1] > 0.0
        hm = jnp.where(mask, h, neg_inf)
        flat_ref[g:g + 1, :] = jnp.max(hm, axis=0, keepdims=True)


_final_pass = pl.pallas_call(
    _final_body,
    out_shape=[
        jax.ShapeDtypeStruct((N, D), jnp.float32),
        jax.ShapeDtypeStruct((G, D), jnp.float32),
    ],
)


def kernel(inputs, edge_index, batch, edge_weight, W, b,
           gn_weight, gn_bias, gn_mean_scale):
    src = edge_index[0].reshape(NW, NCHUNK, CH)
    dst = edge_index[1].reshape(NW, NCHUNK, CH)
    wre = edge_weight.reshape(NW, NCHUNK, CH)
    zeros_n = jnp.zeros((N,), jnp.float32)
    zeros_nd = jnp.zeros((N, D), jnp.float32)

    deg_part = _deg_pass(dst, wre, zeros_n)
    degt = deg_part.T

    z, zsl, dis = _prep_pass(inputs, W, degt)

    conv_part = _edge_pass(src, dst, wre, z, dis.reshape(N), zeros_nd)

    h, flat = _final_pass(
        conv_part, zsl,
        b.reshape(1, D), gn_weight.reshape(1, D), gn_bias.reshape(1, D),
        gn_mean_scale.reshape(1, D),
        batch.reshape(1, N), batch.reshape(N, 1),
    )
    return (h, flat, edge_index, edge_weight, batch)


# speedup vs baseline: 20.0550x; 20.0550x over previous
import functools

import jax
import jax.numpy as jnp
from jax import lax
from jax.experimental import pallas as pl
from jax.experimental.pallas import tpu as pltpu
from jax.experimental.pallas import tpu_sc as plsc

N = 10000
E = 320000
D = 128
G = 16
EPS = 1e-5

NC = 2
NS = 16
NW = NC * NS
EW = E // NW
CH = 80
NCHUNK = EW // CH
NGRP = 5
GCH = NCHUNK // NGRP
NPAD = 10240
SPT = NPAD // NS

def _deg_body(dst_hbm, w_hbm, zeros_hbm, out_hbm, dst_v, w_v, deg_sh):
    cid = lax.axis_index("c")
    sid = lax.axis_index("s")
    wid = cid * NS + sid

    @pl.when(sid == 0)
    def _():
        pltpu.sync_copy(zeros_hbm, deg_sh)

    plsc.subcore_barrier()

    pltpu.sync_copy(dst_hbm.at[wid], dst_v)
    pltpu.sync_copy(w_hbm.at[wid], w_v)

    def chunk(j, carry):
        pltpu.sync_copy(w_v.at[j], deg_sh.at[dst_v.at[j]], add=True)
        return carry

    lax.fori_loop(0, NCHUNK, chunk, 0)
    plsc.subcore_barrier()

    @pl.when(sid == 0)
    def _():
        pltpu.sync_copy(deg_sh, out_hbm.at[cid])


@functools.cache
def _deg_pass():
    mesh = plsc.VectorSubcoreMesh(core_axis_name="c", subcore_axis_name="s",
                                  num_cores=NC, num_subcores=NS)
    return pl.kernel(
        _deg_body,
        out_type=jax.ShapeDtypeStruct((NC, N), jnp.float32),
        mesh=mesh,
        scratch_types=[
            pltpu.VMEM((NCHUNK, CH), jnp.int32),
            pltpu.VMEM((NCHUNK, CH), jnp.float32),
            pltpu.MemorySpace.VMEM_SHARED((N,), jnp.float32),
        ],
    )


def _prep_body(x_ref, w_ref, degt_ref, z_ref, zsl_ref, dis_ref):
    deg = degt_ref[:, 0:1] + degt_ref[:, 1:2] + 1.0
    safe = jnp.where(deg > 0.0, deg, 1.0)
    dis = jnp.where(deg > 0.0, lax.rsqrt(safe), 0.0)
    x = jnp.dot(x_ref[...], w_ref[...], preferred_element_type=jnp.float32)
    z = x * dis
    z_ref[...] = z
    zsl_ref[...] = z * dis
    dis_ref[...] = dis


_prep_pass = pl.pallas_call(
    _prep_body,
    out_shape=[
        jax.ShapeDtypeStruct((N, D), jnp.float32),
        jax.ShapeDtypeStruct((N, D), jnp.float32),
        jax.ShapeDtypeStruct((N, 1), jnp.float32),
    ],
)


def _edge_body(src_hbm, dst_hbm, w_hbm, z_hbm, dis_hbm, zeros_hbm, out_hbm,
               src_v, dst_v, w_v, dis_v, rows_v, s_v, conv_sh, sem):
    cid = lax.axis_index("c")
    sid = lax.axis_index("s")
    wid = cid * NS + sid

    pltpu.sync_copy(zeros_hbm.at[pl.ds(sid * SPT, SPT)],
                    conv_sh.at[pl.ds(sid * SPT, SPT)])
    pltpu.sync_copy(dis_hbm, dis_v)
    plsc.subcore_barrier()

    def group(g, carry0):
        pltpu.sync_copy(src_hbm.at[wid].at[g], src_v)
        pltpu.sync_copy(dst_hbm.at[wid].at[g], dst_v)
        pltpu.sync_copy(w_hbm.at[wid].at[g], w_v)

        def chunk(j, carry):
            pltpu.async_copy(z_hbm.at[src_v.at[j]], rows_v, sem).wait()
            for v in range(CH // 16):
                sl = pl.ds(v * 16, 16)
                dstv = dst_v[j, sl]
                disg = plsc.load_gather(dis_v, [dstv])
                s_v[sl] = w_v[j, sl] * disg

            def rowloop(r0, carry2):
                svec = s_v[pl.ds(r0 * 16, 16)]
                for u in range(16):
                    sc = svec[u]
                    r = r0 * 16 + u
                    for cv in range(D // 16):
                        csl = pl.ds(cv * 16, 16)
                        rows_v[r, csl] = rows_v[r, csl] * sc
                return carry2

            lax.fori_loop(0, CH // 16, rowloop, 0)
            pltpu.sync_copy(rows_v, conv_sh.at[dst_v.at[j]], add=True)
            return carry

        lax.fori_loop(0, GCH, chunk, 0)
        return carry0

    lax.fori_loop(0, NGRP, group, 0)
    plsc.subcore_barrier()

    pltpu.sync_copy(conv_sh.at[pl.ds(sid * SPT, SPT)],
                    out_hbm.at[cid].at[pl.ds(sid * SPT, SPT)])


@functools.cache
def _edge_pass():
    mesh = plsc.VectorSubcoreMesh(core_axis_name="c", subcore_axis_name="s",
                                  num_cores=NC, num_subcores=NS)
    return pl.kernel(
        _edge_body,
        out_type=jax.ShapeDtypeStruct((NC, NPAD, D), jnp.float32),
        mesh=mesh,
        compiler_params=pltpu.CompilerParams(needs_layout_passes=False),
        scratch_types=[
            pltpu.VMEM((GCH, CH), jnp.int32),
            pltpu.VMEM((GCH, CH), jnp.int32),
            pltpu.VMEM((GCH, CH), jnp.float32),
            pltpu.VMEM((N,), jnp.float32),
            pltpu.VMEM((CH, D), jnp.float32),
            pltpu.VMEM((CH,), jnp.float32),
            pltpu.MemorySpace.VMEM_SHARED((NPAD, D), jnp.float32),
            pltpu.SemaphoreType.DMA,
        ],
    )


def _final_body(convp_ref, zsl_ref, b_ref, gnw_ref, gnb_ref, gnms_ref,
                batchr_ref, batchc_ref, h_ref, flat_ref):
    conv = convp_ref[0] + convp_ref[1] + zsl_ref[...] + b_ref[...]
    ms = gnms_ref[...]

    gids = lax.broadcasted_iota(jnp.int32, (G, N), 0)
    onehot = (batchr_ref[...] == gids).astype(jnp.float32)
    counts = jnp.sum(onehot, axis=1, keepdims=True)
    counts = jnp.maximum(counts, 1.0)
    s1 = jnp.dot(onehot, conv, preferred_element_type=jnp.float32)
    s2 = jnp.dot(onehot, conv * conv, preferred_element_type=jnp.float32)
    mean = s1 / counts
    msq = s2 / counts
    var = msq - (2.0 * ms - ms * ms) * mean * mean
    rstd = lax.rsqrt(var + EPS)
    scale = gnw_ref[...] * rstd
    shift = gnb_ref[...] - ms * mean * scale

    gcols = lax.broadcasted_iota(jnp.int32, (N, G), 1)
    onehot_t = (batchc_ref[...] == gcols).astype(jnp.float32)
    scale_full = jnp.dot(onehot_t, scale, preferred_element_type=jnp.float32)
    shift_full = jnp.dot(onehot_t, shift, preferred_element_type=jnp.float32)
    h = jnp.maximum(conv * scale_full + shift_full, 0.0)
    h_ref[...] = h

    neg_inf = jnp.float32(-jnp.inf)
    for g in range(G):
        mask = onehot_t[:, g:g + 1] > 0.0
        hm = jnp.where(mask, h, neg_inf)
        flat_ref[g:g + 1, :] = jnp.max(hm, axis=0, keepdims=True)


_final_pass = pl.pallas_call(
    _final_body,
    out_shape=[
        jax.ShapeDtypeStruct((N, D), jnp.float32),
        jax.ShapeDtypeStruct((G, D), jnp.float32),
    ],
)


def kernel(inputs, edge_index, batch, edge_weight, W, b,
           gn_weight, gn_bias, gn_mean_scale):
    src = edge_index[0].reshape(NW, NGRP, GCH, CH)
    dst = edge_index[1].reshape(NW, NGRP, GCH, CH)
    wre = edge_weight.reshape(NW, NGRP, GCH, CH)
    zeros_n = jnp.zeros((N,), jnp.float32)
    zeros_nd = jnp.zeros((NPAD, D), jnp.float32)

    deg_part = _deg_pass()(dst.reshape(NW, NCHUNK, CH),
                           wre.reshape(NW, NCHUNK, CH), zeros_n)
    degt = deg_part.T

    z, zsl, dis = _prep_pass(inputs, W, degt)

    conv_part = _edge_pass()(src, dst, wre, z, dis.reshape(N), zeros_nd)
    conv_part = conv_part[:, :N, :]

    h, flat = _final_pass(
        conv_part, zsl,
        b.reshape(1, D), gn_weight.reshape(1, D), gn_bias.reshape(1, D),
        gn_mean_scale.reshape(1, D),
        batch.reshape(1, N), batch.reshape(N, 1),
    )
    return (h, flat, edge_index, edge_weight, batch)
